# baseline (speedup 1.0000x reference)
import jax
import jax.numpy as jnp
from jax import lax
from jax.experimental import pallas as pl
from jax.experimental.pallas import tpu as pltpu
from jax.experimental.pallas import tpu_sc as plsc

B, A = 16384, 200
L = 16
NC, NS = 2, 16
NW = NC * NS
BPW = B // NW
GROUPS = BPW // L
ATILES = A // 8


def _gather_tab(tab, sv):
    return lax.gather(
        tab, sv[:, None],
        lax.GatherDimensionNumbers(
            offset_dims=(), collapsed_slice_dims=(0,),
            start_index_map=(0,)),
        slice_sizes=(1,),
        mode=lax.GatherScatterMode.PROMISE_IN_BOUNDS)


def _body(spt_hbm, table_hbm, outsp_hbm, out_hbm,
          buf, tab_v, out_v, sem, semw):
    wid = lax.axis_index("s") * NC + lax.axis_index("c")
    b0 = wid * BPW

    pltpu.sync_copy(table_hbm, tab_v.at[pl.ds(0, 7)])
    tab = tab_v[...]

    copies = [
        pltpu.async_copy(
            spt_hbm.at[pl.ds(at * 8, 8), pl.ds(b0, BPW)],
            buf.at[pl.ds(at * 8, 8), :], sem)
        for at in range(ATILES)
    ]
    wbs = []
    for at in range(ATILES):
        copies[at].wait()
        wbs.append(pltpu.async_copy(
            buf.at[pl.ds(at * 8, 8), :],
            outsp_hbm.at[pl.ds(at * 8, 8), pl.ds(b0, BPW)], semw))

    zero = jnp.zeros((L,), jnp.float32)
    for g in range(0, GROUPS, 2):
        G = g * L

        @plsc.parallel_loop(0, A, 1, unroll=10,
                            carry=(zero, zero, zero, zero))
        def acc_loop(a, accs, G=G):
            a0, a1, b0_, b1 = accs
            sva = buf[a, pl.ds(G, L)]
            svb = buf[a, pl.ds(G + L, L)]
            return (a1, a0 + _gather_tab(tab, sva),
                    b1, b0_ + _gather_tab(tab, svb))

        a0, a1, b0_, b1 = acc_loop
        out_v[pl.ds(G, L)] = a0 + a1
        out_v[pl.ds(G + L, L)] = b0_ + b1

    pltpu.sync_copy(out_v, out_hbm.at[pl.ds(b0, BPW)])
    for w in wbs:
        w.wait()


_mesh = plsc.VectorSubcoreMesh(core_axis_name="c", subcore_axis_name="s",
                               num_cores=NC, num_subcores=NS)

_sc_call = pl.kernel(
    _body,
    out_type=(jax.ShapeDtypeStruct((A, B), jnp.int32),
              jax.ShapeDtypeStruct((B,), jnp.float32)),
    mesh=_mesh,
    scratch_types=[
        pltpu.VMEM((A, BPW), jnp.int32),
        pltpu.VMEM((L,), jnp.float32),
        pltpu.VMEM((BPW,), jnp.float32),
        pltpu.SemaphoreType.DMA,
        pltpu.SemaphoreType.DMA,
    ],
    compiler_params=pltpu.CompilerParams(use_tc_tiling_on_sc=True,
                                         needs_layout_passes=False),
    name="energy_shifter_sc",
)


def kernel(species, energies, self_energies, intercept):
    tab7 = self_energies.astype(jnp.float32) + intercept / A
    spt_out, out = _sc_call(species.T, tab7)
    return (spt_out.T, out)


# speedup vs baseline: 1.1185x; 1.1185x over previous
import jax
import jax.numpy as jnp
from jax import lax
from jax.experimental import pallas as pl
from jax.experimental.pallas import tpu as pltpu
from jax.experimental.pallas import tpu_sc as plsc

B, A = 16384, 200
L = 16
NC, NS = 2, 16
NW = NC * NS
BPW = B // NW
GROUPS = BPW // L
ATILES = A // 8


def _gather_tab(tab, sv):
    return lax.gather(
        tab, sv[:, None],
        lax.GatherDimensionNumbers(
            offset_dims=(), collapsed_slice_dims=(0,),
            start_index_map=(0,)),
        slice_sizes=(1,),
        mode=lax.GatherScatterMode.PROMISE_IN_BOUNDS)


def _body(spt_hbm, table_hbm, outsp_hbm, out_hbm,
          buf, tab_v, out_v, sem, semw):
    wid = lax.axis_index("s") * NC + lax.axis_index("c")
    b0 = wid * BPW

    pltpu.sync_copy(table_hbm, tab_v.at[pl.ds(0, 7)])
    tab = tab_v[...]

    copies = [
        pltpu.async_copy(
            spt_hbm.at[pl.ds(at * 8, 8), pl.ds(b0, BPW)],
            buf.at[pl.ds(at * 8, 8), :], sem)
        for at in range(ATILES)
    ]
    for c in copies:
        c.wait()
    wbs = [
        pltpu.async_copy(
            buf.at[pl.ds(at * 8, 8), :],
            outsp_hbm.at[pl.ds(at * 8, 8), pl.ds(b0, BPW)], semw)
        for at in range(ATILES)
    ]

    zero = jnp.zeros((L,), jnp.float32)
    for g in range(0, GROUPS, 2):
        G = g * L

        @plsc.parallel_loop(0, A, 1, unroll=8,
                            carry=(zero, zero, zero, zero))
        def acc_loop(a, accs, G=G):
            a0, a1, b0_, b1 = accs
            sva = buf[a, pl.ds(G, L)]
            svb = buf[a, pl.ds(G + L, L)]
            return (a1, a0 + _gather_tab(tab, sva),
                    b1, b0_ + _gather_tab(tab, svb))

        a0, a1, b0_, b1 = acc_loop
        out_v[pl.ds(G, L)] = a0 + a1
        out_v[pl.ds(G + L, L)] = b0_ + b1

    pltpu.sync_copy(out_v, out_hbm.at[pl.ds(b0, BPW)])
    for w in wbs:
        w.wait()


_mesh = plsc.VectorSubcoreMesh(core_axis_name="c", subcore_axis_name="s",
                               num_cores=NC, num_subcores=NS)

_sc_call = pl.kernel(
    _body,
    out_type=(jax.ShapeDtypeStruct((A, B), jnp.int32),
              jax.ShapeDtypeStruct((B,), jnp.float32)),
    mesh=_mesh,
    scratch_types=[
        pltpu.VMEM((A, BPW), jnp.int32),
        pltpu.VMEM((L,), jnp.float32),
        pltpu.VMEM((BPW,), jnp.float32),
        pltpu.SemaphoreType.DMA,
        pltpu.SemaphoreType.DMA,
    ],
    compiler_params=pltpu.CompilerParams(use_tc_tiling_on_sc=True,
                                         needs_layout_passes=False),
    name="energy_shifter_sc",
)


def kernel(species, energies, self_energies, intercept):
    tab7 = self_energies.astype(jnp.float32) + intercept / A
    spt_out, out = _sc_call(species.T, tab7)
    return (spt_out.T, out)


# speedup vs baseline: 1.1419x; 1.0209x over previous
import jax
import jax.numpy as jnp
from jax import lax
from jax.experimental import pallas as pl
from jax.experimental.pallas import tpu as pltpu
from jax.experimental.pallas import tpu_sc as plsc

B, A = 16384, 200
L = 16
NC, NS = 2, 16
NW = NC * NS
BPW = B // NW
GROUPS = BPW // L
ATILES = A // 8


def _gather_tab(tab, sv):
    return lax.gather(
        tab, sv[:, None],
        lax.GatherDimensionNumbers(
            offset_dims=(), collapsed_slice_dims=(0,),
            start_index_map=(0,)),
        slice_sizes=(1,),
        mode=lax.GatherScatterMode.PROMISE_IN_BOUNDS)


def _body(spt_hbm, table_hbm, outsp_hbm, out_hbm,
          buf, tab_v, out_v, sem, semw):
    wid = lax.axis_index("s") * NC + lax.axis_index("c")
    b0 = wid * BPW

    pltpu.sync_copy(table_hbm, tab_v.at[pl.ds(0, 7)])
    tab = tab_v[...]

    copies = [
        pltpu.async_copy(
            spt_hbm.at[pl.ds(ch * 40, 40), pl.ds(b0, BPW)],
            buf.at[pl.ds(ch * 40, 40), :], sem)
        for ch in range(5)
    ]
    for c in copies:
        c.wait()
    wbs = [
        pltpu.async_copy(
            buf.at[pl.ds(ch * 40, 40), :],
            outsp_hbm.at[pl.ds(ch * 40, 40), pl.ds(b0, BPW)], semw)
        for ch in range(5)
    ]

    zero = jnp.zeros((L,), jnp.float32)
    for g in range(0, GROUPS, 2):
        G = g * L

        @plsc.parallel_loop(0, A, 1, unroll=8,
                            carry=(zero, zero, zero, zero))
        def acc_loop(a, accs, G=G):
            a0, a1, b0_, b1 = accs
            sva = buf[a, pl.ds(G, L)]
            svb = buf[a, pl.ds(G + L, L)]
            return (a1, a0 + _gather_tab(tab, sva),
                    b1, b0_ + _gather_tab(tab, svb))

        a0, a1, b0_, b1 = acc_loop
        out_v[pl.ds(G, L)] = a0 + a1
        out_v[pl.ds(G + L, L)] = b0_ + b1

    pltpu.sync_copy(out_v, out_hbm.at[pl.ds(b0, BPW)])
    for w in wbs:
        w.wait()


_mesh = plsc.VectorSubcoreMesh(core_axis_name="c", subcore_axis_name="s",
                               num_cores=NC, num_subcores=NS)

_sc_call = pl.kernel(
    _body,
    out_type=(jax.ShapeDtypeStruct((A, B), jnp.int32),
              jax.ShapeDtypeStruct((B,), jnp.float32)),
    mesh=_mesh,
    scratch_types=[
        pltpu.VMEM((A, BPW), jnp.int32),
        pltpu.VMEM((L,), jnp.float32),
        pltpu.VMEM((BPW,), jnp.float32),
        pltpu.SemaphoreType.DMA,
        pltpu.SemaphoreType.DMA,
    ],
    compiler_params=pltpu.CompilerParams(use_tc_tiling_on_sc=True,
                                         needs_layout_passes=False),
    name="energy_shifter_sc",
)


def kernel(species, energies, self_energies, intercept):
    tab7 = self_energies.astype(jnp.float32) + intercept / A
    spt_out, out = _sc_call(species.T, tab7)
    return (spt_out.T, out)


# speedup vs baseline: 1.1458x; 1.0034x over previous
import jax
import jax.numpy as jnp
from jax import lax
from jax.experimental import pallas as pl
from jax.experimental.pallas import tpu as pltpu
from jax.experimental.pallas import tpu_sc as plsc

B, A = 16384, 200
L = 16
NC, NS = 2, 16
NW = NC * NS
BPW = B // NW
GROUPS = BPW // L
ATILES = A // 8


def _gather_tab(tab, sv):
    return lax.gather(
        tab, sv[:, None],
        lax.GatherDimensionNumbers(
            offset_dims=(), collapsed_slice_dims=(0,),
            start_index_map=(0,)),
        slice_sizes=(1,),
        mode=lax.GatherScatterMode.PROMISE_IN_BOUNDS)


def _body(spt_hbm, table_hbm, outsp_hbm, out_hbm,
          buf, tab_v, out_v, sem, semw):
    wid = lax.axis_index("s") * NC + lax.axis_index("c")
    b0 = wid * BPW

    pltpu.sync_copy(table_hbm, tab_v.at[pl.ds(0, 7)])
    tab = tab_v[...]

    copies = [
        pltpu.async_copy(
            spt_hbm.at[pl.ds(ch * 200, 200), pl.ds(b0, BPW)],
            buf.at[pl.ds(ch * 200, 200), :], sem)
        for ch in range(1)
    ]
    for c in copies:
        c.wait()
    wbs = [
        pltpu.async_copy(
            buf.at[pl.ds(ch * 200, 200), :],
            outsp_hbm.at[pl.ds(ch * 200, 200), pl.ds(b0, BPW)], semw)
        for ch in range(1)
    ]

    zero = jnp.zeros((L,), jnp.float32)
    for g in range(0, GROUPS, 2):
        G = g * L

        @plsc.parallel_loop(0, A, 1, unroll=8,
                            carry=(zero, zero, zero, zero))
        def acc_loop(a, accs, G=G):
            a0, a1, b0_, b1 = accs
            sva = buf[a, pl.ds(G, L)]
            svb = buf[a, pl.ds(G + L, L)]
            return (a1, a0 + _gather_tab(tab, sva),
                    b1, b0_ + _gather_tab(tab, svb))

        a0, a1, b0_, b1 = acc_loop
        out_v[pl.ds(G, L)] = a0 + a1
        out_v[pl.ds(G + L, L)] = b0_ + b1

    pltpu.sync_copy(out_v, out_hbm.at[pl.ds(b0, BPW)])
    for w in wbs:
        w.wait()


_mesh = plsc.VectorSubcoreMesh(core_axis_name="c", subcore_axis_name="s",
                               num_cores=NC, num_subcores=NS)

_sc_call = pl.kernel(
    _body,
    out_type=(jax.ShapeDtypeStruct((A, B), jnp.int32),
              jax.ShapeDtypeStruct((B,), jnp.float32)),
    mesh=_mesh,
    scratch_types=[
        pltpu.VMEM((A, BPW), jnp.int32),
        pltpu.VMEM((L,), jnp.float32),
        pltpu.VMEM((BPW,), jnp.float32),
        pltpu.SemaphoreType.DMA,
        pltpu.SemaphoreType.DMA,
    ],
    compiler_params=pltpu.CompilerParams(use_tc_tiling_on_sc=True,
                                         needs_layout_passes=False),
    name="energy_shifter_sc",
)


def kernel(species, energies, self_energies, intercept):
    tab7 = self_energies.astype(jnp.float32) + intercept / A
    spt_out, out = _sc_call(species.T, tab7)
    return (spt_out.T, out)
